# baseline (speedup 1.0000x reference)
import functools

import jax
import jax.numpy as jnp
from jax import lax
from jax.experimental import pallas as pl
from jax.experimental.pallas import tpu as pltpu
from jax.experimental.pallas import tpu_sc as plsc

_NUM_CORES = 2
_NUM_SUBCORES = 16
_NUM_WORKERS = _NUM_CORES * _NUM_SUBCORES

_CHUNK = 128


def _make_gather(total, hidden):
  assert total % (_NUM_WORKERS * 2 * _CHUNK) == 0
  per_worker = total // _NUM_WORKERS
  chunks = per_worker // _CHUNK
  pairs = chunks // 2

  mesh = plsc.VectorSubcoreMesh(
      core_axis_name="c", subcore_axis_name="s",
      num_cores=_NUM_CORES, num_subcores=_NUM_SUBCORES)

  @functools.partial(
      pl.kernel,
      out_type=jax.ShapeDtypeStruct((total, hidden), jnp.float32),
      mesh=mesh,
      scratch_types=[
          pltpu.VMEM((per_worker,), jnp.int32),
          pltpu.VMEM((2, _CHUNK, hidden), jnp.float32),
          pltpu.SemaphoreType.DMA,
          pltpu.SemaphoreType.DMA,
          pltpu.SemaphoreType.DMA,
          pltpu.SemaphoreType.DMA,
      ],
      compiler_params=pltpu.CompilerParams(use_tc_tiling_on_sc=False),
  )
  def gather_kernel(table_hbm, idx_hbm, out_hbm, idx_v, rows_v,
                    gsem0, gsem1, osem0, osem1):
    wid = lax.axis_index("s") * _NUM_CORES + lax.axis_index("c")
    base = wid * per_worker

    pltpu.sync_copy(idx_hbm.at[pl.ds(base, per_worker)], idx_v)

    gsems = (gsem0, gsem1)
    osems = (osem0, osem1)

    def gather_start(j, buf):
      pltpu.async_copy(table_hbm.at[idx_v.at[pl.ds(j * _CHUNK, _CHUNK)]],
                       rows_v.at[buf], gsems[buf])

    def gather_wait(j, buf):
      pltpu.make_async_copy(table_hbm.at[idx_v.at[pl.ds(j * _CHUNK, _CHUNK)]],
                            rows_v.at[buf], gsems[buf]).wait()

    def out_start(j, buf):
      pltpu.async_copy(rows_v.at[buf],
                       out_hbm.at[pl.ds(base + j * _CHUNK, _CHUNK)],
                       osems[buf])

    def out_wait(buf):
      pltpu.make_async_copy(rows_v.at[buf],
                            out_hbm.at[pl.ds(base, _CHUNK)],
                            osems[buf]).wait()

    gather_start(0, 0)

    def body(p, _):
      @pl.when(p >= 1)
      def _():
        out_wait(1)
      gather_start(2 * p + 1, 1)
      gather_wait(2 * p, 0)
      out_start(2 * p, 0)
      @pl.when(p < pairs - 1)
      def _():
        out_wait(0)
        gather_start(2 * p + 2, 0)
      gather_wait(2 * p + 1, 1)
      out_start(2 * p + 1, 1)
      return ()

    lax.fori_loop(0, pairs, body, ())

    out_wait(0)
    out_wait(1)

  return gather_kernel


def kernel(inputs, embeddings):
  batch, seq = inputs.shape
  hidden = embeddings.shape[1]
  total = batch * seq
  flat_idx = jnp.reshape(inputs.astype(jnp.int32), (total,))
  out = _make_gather(total, hidden)(embeddings, flat_idx)
  return jnp.reshape(out, (batch, seq, hidden))


# speedup vs baseline: 1.0076x; 1.0076x over previous
import functools

import jax
import jax.numpy as jnp
from jax import lax
from jax.experimental import pallas as pl
from jax.experimental.pallas import tpu as pltpu
from jax.experimental.pallas import tpu_sc as plsc

_NUM_CORES = 2
_NUM_SUBCORES = 16
_NUM_WORKERS = _NUM_CORES * _NUM_SUBCORES

_CHUNK = 128


_NBUF = 5
_AHEAD = 3


def _make_gather(total, hidden):
  assert total % (_NUM_WORKERS * _CHUNK) == 0
  per_worker = total // _NUM_WORKERS
  chunks = per_worker // _CHUNK
  assert chunks % _NBUF == 0 and chunks >= _NBUF
  groups = chunks // _NBUF

  mesh = plsc.VectorSubcoreMesh(
      core_axis_name="c", subcore_axis_name="s",
      num_cores=_NUM_CORES, num_subcores=_NUM_SUBCORES)

  @functools.partial(
      pl.kernel,
      out_type=jax.ShapeDtypeStruct((total, hidden), jnp.float32),
      mesh=mesh,
      scratch_types=[
          pltpu.VMEM((per_worker,), jnp.int32),
          pltpu.VMEM((_NBUF, _CHUNK, hidden), jnp.float32),
          [pltpu.SemaphoreType.DMA] * _NBUF,
          [pltpu.SemaphoreType.DMA] * _NBUF,
      ],
      compiler_params=pltpu.CompilerParams(use_tc_tiling_on_sc=False),
  )
  def gather_kernel(table_hbm, idx_hbm, out_hbm, idx_v, rows_v, gsems, osems):
    wid = lax.axis_index("s") * _NUM_CORES + lax.axis_index("c")
    base = wid * per_worker

    pltpu.sync_copy(idx_hbm.at[pl.ds(base, per_worker)], idx_v)

    def gather_start(j, buf):
      pltpu.async_copy(table_hbm.at[idx_v.at[pl.ds(j * _CHUNK, _CHUNK)]],
                       rows_v.at[buf], gsems[buf])

    def gather_wait(j, buf):
      pltpu.make_async_copy(table_hbm.at[idx_v.at[pl.ds(j * _CHUNK, _CHUNK)]],
                            rows_v.at[buf], gsems[buf]).wait()

    def out_start(j, buf):
      pltpu.async_copy(rows_v.at[buf],
                       out_hbm.at[pl.ds(base + j * _CHUNK, _CHUNK)],
                       osems[buf])

    def out_wait(buf):
      pltpu.make_async_copy(rows_v.at[buf],
                            out_hbm.at[pl.ds(base, _CHUNK)],
                            osems[buf]).wait()

    for k in range(_AHEAD):
      gather_start(k, k)

    def body(g, _):
      for b in range(_NBUF):
        j = g * _NBUF + b
        nb = (b + _AHEAD) % _NBUF

        @pl.when(j + _AHEAD < chunks)
        def _():
          @pl.when(j + _AHEAD >= _NBUF)
          def _():
            out_wait(nb)
          gather_start(j + _AHEAD, nb)

        gather_wait(j, b)
        out_start(j, b)
      return ()

    lax.fori_loop(0, groups, body, ())

    for c in range(chunks - _NBUF, chunks):
      out_wait(c % _NBUF)

  return gather_kernel


def kernel(inputs, embeddings):
  batch, seq = inputs.shape
  hidden = embeddings.shape[1]
  total = batch * seq
  flat_idx = jnp.reshape(inputs.astype(jnp.int32), (total,))
  out = _make_gather(total, hidden)(embeddings, flat_idx)
  return jnp.reshape(out, (batch, seq, hidden))
